# Initial kernel scaffold; baseline (speedup 1.0000x reference)
import functools

import jax
import jax.numpy as jnp
from jax import lax
from jax.experimental import pallas as pl
from jax.experimental.pallas import tpu as pltpu
from jax.experimental.pallas import tpu_sc as plsc

N_NODES = 10000
N_EDGES = 10000
NNZ = 160000
D = 256
DH = 128
NC = 2
NS = 16
L = 16

ROWS_PER_TILE = N_NODES // NS
NNZ_PER_TILE = NNZ // NS
CH = 80
NCHUNK = NNZ_PER_TILE // CH
RCH = 125
NRCH = ROWS_PER_TILE // RCH

_MESH = plsc.VectorSubcoreMesh(
    core_axis_name="c", subcore_axis_name="s", num_cores=NC, num_subcores=NS)


def _make_sc_layer(compute_hists):
    out_type = [jax.ShapeDtypeStruct((N_NODES, DH), jnp.float32),
                jax.ShapeDtypeStruct((N_NODES, DH), jnp.float32)]
    if compute_hists:
        out_type += [jax.ShapeDtypeStruct((N_NODES, L), jnp.float32),
                     jax.ShapeDtypeStruct((N_EDGES, L), jnp.float32)]
    scratch = []
    if compute_hists:
        scratch += [pltpu.VMEM_SHARED((N_EDGES, L), jnp.float32),
                    pltpu.VMEM_SHARED((N_NODES, L), jnp.float32)]
    scratch += [
        pltpu.VMEM_SHARED((N_NODES, DH), jnp.float32),
        pltpu.VMEM((NCHUNK, CH), jnp.int32),
        pltpu.VMEM((NCHUNK, CH), jnp.int32),
        pltpu.VMEM((CH, DH), jnp.float32),
        pltpu.VMEM((RCH, DH), jnp.float32),
        pltpu.VMEM((RCH, L), jnp.float32),
        pltpu.VMEM((CH, L), jnp.float32),
        pltpu.VMEM((RCH, DH), jnp.float32),
        pltpu.VMEM((ROWS_PER_TILE, L), jnp.float32),
        pltpu.SemaphoreType.DMA,
    ]

    def body(*refs):
        if compute_hists:
            (xl_a, xl_b, nidx, eidx,
             out_a, out_b, nhist_out, ehist_out,
             hist_e, hist_n,
             acc, idxn, idxe, rows, rbuf, hbuf, ones, zbuf, z16, sem) = refs
            ehist_in = None
        else:
            (xl_a, xl_b, nidx, eidx, ehist_in,
             out_a, out_b,
             acc, idxn, idxe, rows, rbuf, hbuf, ones, zbuf, z16, sem) = refs
            hist_e = hist_n = None
        c = lax.axis_index("c")
        s = lax.axis_index("s")

        def fill_const(ref, nrows, val):
            v = jnp.full((L,), val, jnp.float32)
            ncol = ref.shape[1] // L

            def fb(r, carry):
                for j in range(ncol):
                    ref[r, pl.ds(j * L, L)] = v
                return carry
            lax.fori_loop(0, nrows, fb, 0)

        def zero_acc():
            for k in range(NRCH):
                pltpu.sync_copy(
                    zbuf, acc.at[pl.ds(s * ROWS_PER_TILE + k * RCH, RCH)])

        def hist_pass(idx2d, hist):
            def hb(i, carry):
                pltpu.sync_copy(ones, hist.at[idx2d.at[i]], add=True)
                return carry
            lax.fori_loop(0, NCHUNK, hb, 0)

        def seg_pass(src_hbm, gidx, sidx):
            def sb(i, carry):
                pltpu.async_copy(src_hbm.at[gidx.at[i]], rows, sem).wait()
                pltpu.sync_copy(rows, acc.at[sidx.at[i]], add=True)
                return carry
            lax.fori_loop(0, NCHUNK, sb, 0)

        def scale_write(hist_ref, out_hbm):
            for k in range(NRCH):
                r0 = s * ROWS_PER_TILE + k * RCH
                pltpu.sync_copy(acc.at[pl.ds(r0, RCH)], rbuf)
                pltpu.sync_copy(hist_ref.at[pl.ds(r0, RCH)], hbuf)

                def rb(r, carry):
                    hv = hbuf[r]
                    inv = jnp.where(hv > 0.0, 1.0 / hv, 0.0)
                    for j in range(DH // L):
                        sl = pl.ds(j * L, L)
                        rbuf[r, sl] = rbuf[r, sl] * inv
                    return carry
                lax.fori_loop(0, RCH, rb, 0)
                pltpu.sync_copy(rbuf, out_hbm.at[pl.ds(r0, RCH)])

        def plain_write(out_hbm):
            for k in range(NRCH):
                r0 = s * ROWS_PER_TILE + k * RCH
                pltpu.sync_copy(acc.at[pl.ds(r0, RCH)], rbuf)
                pltpu.sync_copy(rbuf, out_hbm.at[pl.ds(r0, RCH)])

        fill_const(zbuf, RCH, 0.0)
        pltpu.sync_copy(nidx.at[s], idxn)
        pltpu.sync_copy(eidx.at[s], idxe)
        stripe = pl.ds(s * ROWS_PER_TILE, ROWS_PER_TILE)

        if compute_hists:
            fill_const(ones, CH, 1.0)
            fill_const(z16, ROWS_PER_TILE, 0.0)
            pltpu.sync_copy(z16, hist_e.at[stripe])

            @pl.when(c == 0)
            def _():
                pltpu.sync_copy(z16, hist_n.at[stripe])
            plsc.subcore_barrier()
            hist_pass(idxe, hist_e)

            @pl.when(c == 0)
            def _():
                hist_pass(idxn, hist_n)
            plsc.subcore_barrier()

            @pl.when(c == 0)
            def _():
                pltpu.sync_copy(hist_e.at[stripe], z16)
                pltpu.sync_copy(z16, ehist_out.at[stripe])
                pltpu.sync_copy(hist_n.at[stripe], z16)
                pltpu.sync_copy(z16, nhist_out.at[stripe])
            bscale = hist_e
        else:
            bscale = ehist_in

        zero_acc()
        plsc.subcore_barrier()

        def run_half(xl, out_h):
            seg_pass(xl, idxn, idxe)
            plsc.subcore_barrier()
            scale_write(bscale, out_h)
            zero_acc()
            plsc.subcore_barrier()
            seg_pass(out_h, idxe, idxn)
            plsc.subcore_barrier()
            plain_write(out_h)

        @pl.when(c == 0)
        def _():
            run_half(xl_a, out_a)

        @pl.when(c == 1)
        def _():
            run_half(xl_b, out_b)

    return pl.kernel(body, out_type=out_type, mesh=_MESH,
                     scratch_types=scratch)


_sc_layer0 = _make_sc_layer(True)
_sc_layer1 = _make_sc_layer(False)

_BLK = 1000
_GRID = (N_NODES // _BLK,)


def _dot_t(x, w):
    return lax.dot_general(x, w, (((1,), (1,)), ((), ())),
                           preferred_element_type=jnp.float32)


def _tc_lin(X, W):
    def body(x_ref, w_ref, oa_ref, ob_ref):
        y = _dot_t(x_ref[...], w_ref[...])
        oa_ref[...] = y[:, :DH]
        ob_ref[...] = y[:, DH:]

    return pl.pallas_call(
        body,
        grid=_GRID,
        in_specs=[pl.BlockSpec((_BLK, D), lambda i: (i, 0)),
                  pl.BlockSpec((D, D), lambda i: (0, 0))],
        out_specs=[pl.BlockSpec((_BLK, DH), lambda i: (i, 0)),
                   pl.BlockSpec((_BLK, DH), lambda i: (i, 0))],
        out_shape=[jax.ShapeDtypeStruct((N_NODES, DH), jnp.float32)] * 2,
    )(X, W)


def _epilogue(ya, yb, deg16, b, g, beta):
    h = jnp.concatenate([ya, yb], axis=1)
    deg = deg16[:, :1]
    dinv = jnp.where(deg > 0.0, 1.0 / deg, 0.0)
    h = h * dinv + b
    h = jnp.where(h >= 0.0, h, 0.01 * h)
    mu = jnp.mean(h, axis=1, keepdims=True)
    d = h - mu
    var = jnp.mean(d * d, axis=1, keepdims=True)
    return d * lax.rsqrt(var + 1e-5) * g + beta


def _tc_mid(ya, yb, nhist, b, g, beta, W):
    def body(ya_ref, yb_ref, nh_ref, b_ref, g_ref, be_ref, w_ref,
             h_ref, oa_ref, ob_ref):
        hn = _epilogue(ya_ref[...], yb_ref[...], nh_ref[...],
                       b_ref[...], g_ref[...], be_ref[...])
        h_ref[...] = hn
        y = _dot_t(hn, w_ref[...])
        oa_ref[...] = y[:, :DH]
        ob_ref[...] = y[:, DH:]

    vec = pl.BlockSpec((1, D), lambda i: (0, 0))
    return pl.pallas_call(
        body,
        grid=_GRID,
        in_specs=[pl.BlockSpec((_BLK, DH), lambda i: (i, 0)),
                  pl.BlockSpec((_BLK, DH), lambda i: (i, 0)),
                  pl.BlockSpec((_BLK, L), lambda i: (i, 0)),
                  vec, vec, vec,
                  pl.BlockSpec((D, D), lambda i: (0, 0))],
        out_specs=[pl.BlockSpec((_BLK, D), lambda i: (i, 0)),
                   pl.BlockSpec((_BLK, DH), lambda i: (i, 0)),
                   pl.BlockSpec((_BLK, DH), lambda i: (i, 0))],
        out_shape=[jax.ShapeDtypeStruct((N_NODES, D), jnp.float32),
                   jax.ShapeDtypeStruct((N_NODES, DH), jnp.float32),
                   jax.ShapeDtypeStruct((N_NODES, DH), jnp.float32)],
    )(ya, yb, nhist, b, g, beta, W)


def _tc_final(ya, yb, nhist, b, g, beta, X, h1):
    def body(ya_ref, yb_ref, nh_ref, b_ref, g_ref, be_ref, x_ref, h1_ref,
             o_ref):
        h2 = _epilogue(ya_ref[...], yb_ref[...], nh_ref[...],
                       b_ref[...], g_ref[...], be_ref[...])
        o_ref[...] = (x_ref[...] + h1_ref[...] + h2) * (1.0 / 3.0)

    vec = pl.BlockSpec((1, D), lambda i: (0, 0))
    return pl.pallas_call(
        body,
        grid=_GRID,
        in_specs=[pl.BlockSpec((_BLK, DH), lambda i: (i, 0)),
                  pl.BlockSpec((_BLK, DH), lambda i: (i, 0)),
                  pl.BlockSpec((_BLK, L), lambda i: (i, 0)),
                  vec, vec, vec,
                  pl.BlockSpec((_BLK, D), lambda i: (i, 0)),
                  pl.BlockSpec((_BLK, D), lambda i: (i, 0))],
        out_specs=pl.BlockSpec((_BLK, D), lambda i: (i, 0)),
        out_shape=jax.ShapeDtypeStruct((N_NODES, D), jnp.float32),
    )(ya, yb, nhist, b, g, beta, X, h1)


def kernel(X, A, W0, b0, g0, beta0, W1, b1, g1, beta1):
    nidx = A[0].reshape(NS, NCHUNK, CH)
    eidx = A[1].reshape(NS, NCHUNK, CH)
    b0r, g0r, be0r = (v.reshape(1, D) for v in (b0, g0, beta0))
    b1r, g1r, be1r = (v.reshape(1, D) for v in (b1, g1, beta1))

    xa0, xb0 = _tc_lin(X, W0)
    oa0, ob0, nhist, ehist = _sc_layer0(xa0, xb0, nidx, eidx)
    h1, xa1, xb1 = _tc_mid(oa0, ob0, nhist, b0r, g0r, be0r, W1)
    oa1, ob1 = _sc_layer1(xa1, xb1, nidx, eidx, ehist)
    return _tc_final(oa1, ob1, nhist, b1r, g1r, be1r, X, h1)


# speedup vs baseline: 4.3550x; 4.3550x over previous
import functools

import jax
import jax.numpy as jnp
from jax import lax
from jax.experimental import pallas as pl
from jax.experimental.pallas import tpu as pltpu
from jax.experimental.pallas import tpu_sc as plsc

N_NODES = 10000
N_EDGES = 10000
NNZ = 160000
D = 256
DH = 128
NC = 2
NS = 16
L = 16

NPAD = 10240
ROWS_PER_TILE = NPAD // NS
NNZ_PER_TILE = NNZ // NS
CH = 80
NCHUNK = NNZ_PER_TILE // CH
RCH = 128
NRCH = ROWS_PER_TILE // RCH

_MESH = plsc.VectorSubcoreMesh(
    core_axis_name="c", subcore_axis_name="s", num_cores=NC, num_subcores=NS)


def _make_sc_layer(compute_degs):
    out_type = [jax.ShapeDtypeStruct((NPAD, DH), jnp.float32),
                jax.ShapeDtypeStruct((NPAD, DH), jnp.float32)]
    if compute_degs:
        out_type += [jax.ShapeDtypeStruct((NC, NPAD, DH), jnp.float32),
                     jax.ShapeDtypeStruct((NC, NPAD, DH), jnp.float32)]
    scratch = [
        pltpu.VMEM_SHARED((NPAD, DH), jnp.float32),
        pltpu.VMEM((CH,), jnp.int32),
        pltpu.VMEM((CH,), jnp.int32),
        pltpu.VMEM((CH, DH), jnp.float32),
        pltpu.VMEM((RCH, DH), jnp.float32),
        pltpu.VMEM((RCH, DH), jnp.float32),
        pltpu.SemaphoreType.DMA,
    ]

    def body(*refs):
        if compute_degs:
            (xl_a, xl_b, nidx, eidx,
             out_a, out_b, ndeg_out, edeg_out,
             acc, idx1, idx2, rows, rbuf, dbuf, sem) = refs
            ndeg, edeg = ndeg_out, edeg_out
        else:
            (xl_a, xl_b, nidx, eidx, ndeg_in, edeg_in,
             out_a, out_b,
             acc, idx1, idx2, rows, rbuf, dbuf, sem) = refs
            ndeg, edeg = ndeg_in, edeg_in
        c = lax.axis_index("c")
        s = lax.axis_index("s")

        def fill_const(ref, nrows, val):
            v = jnp.full((L,), val, jnp.float32)

            def fb(r, carry):
                for j in range(DH // L):
                    ref[r, pl.ds(j * L, L)] = v
                return carry
            lax.fori_loop(0, nrows, fb, 0)

        def zero_acc():
            fill_const(rbuf, RCH, 0.0)
            for k in range(NRCH):
                pltpu.sync_copy(
                    rbuf, acc.at[pl.ds(s * ROWS_PER_TILE + k * RCH, RCH)])

        def hist_pass(idx_hbm):
            def hb(i, carry):
                base = s * NNZ_PER_TILE + i * CH
                pltpu.sync_copy(idx_hbm.at[pl.ds(base, CH)], idx2)
                pltpu.sync_copy(rows, acc.at[idx2], add=True)
                return carry
            lax.fori_loop(0, NCHUNK, hb, 0)

        def dump_deg(table):
            for k in range(NRCH):
                sl = pl.ds(s * ROWS_PER_TILE + k * RCH, RCH)
                pltpu.sync_copy(acc.at[sl], dbuf)
                pltpu.sync_copy(dbuf, table.at[c].at[sl])

        def seg_pass(src_hbm, gidx_hbm, sidx_hbm):
            def sb(i, carry):
                base = s * NNZ_PER_TILE + i * CH
                pltpu.sync_copy(gidx_hbm.at[pl.ds(base, CH)], idx1)
                pltpu.sync_copy(sidx_hbm.at[pl.ds(base, CH)], idx2)
                pltpu.async_copy(src_hbm.at[idx1], rows, sem).wait()
                pltpu.sync_copy(rows, acc.at[idx2], add=True)
                return carry
            lax.fori_loop(0, NCHUNK, sb, 0)

        def scale_write(table, out_hbm):
            for k in range(NRCH):
                sl = pl.ds(s * ROWS_PER_TILE + k * RCH, RCH)
                pltpu.sync_copy(acc.at[sl], rbuf)
                pltpu.sync_copy(table.at[c].at[sl], dbuf)

                def rb(r, carry):
                    dv = dbuf[r, pl.ds(0, L)]
                    inv = jnp.where(dv > 0.0, 1.0 / dv, 0.0)
                    for j in range(DH // L):
                        cs = pl.ds(j * L, L)
                        rbuf[r, cs] = rbuf[r, cs] * inv
                    return carry
                lax.fori_loop(0, RCH, rb, 0)
                pltpu.sync_copy(rbuf, out_hbm.at[sl])

        if compute_degs:
            fill_const(rows, CH, 1.0)
            zero_acc()
            plsc.subcore_barrier()
            hist_pass(eidx)
            plsc.subcore_barrier()
            dump_deg(edeg)
            zero_acc()
            plsc.subcore_barrier()
            hist_pass(nidx)
            plsc.subcore_barrier()
            dump_deg(ndeg)

        zero_acc()
        plsc.subcore_barrier()

        def run_half(xl, out_h):
            seg_pass(xl, nidx, eidx)
            plsc.subcore_barrier()
            scale_write(edeg, out_h)
            zero_acc()
            plsc.subcore_barrier()
            seg_pass(out_h, eidx, nidx)
            plsc.subcore_barrier()
            scale_write(ndeg, out_h)

        @pl.when(c == 0)
        def _():
            run_half(xl_a, out_a)

        @pl.when(c == 1)
        def _():
            run_half(xl_b, out_b)

    return pl.kernel(body, out_type=out_type, mesh=_MESH,
                     scratch_types=scratch)


_sc_layer0 = _make_sc_layer(True)
_sc_layer1 = _make_sc_layer(False)


_BLK = 1000
_GRID = (N_NODES // _BLK,)


def _dot_t(x, w):
    return lax.dot_general(x, w, (((1,), (1,)), ((), ())),
                           preferred_element_type=jnp.float32)


def _tc_lin(X, W):
    def body(x_ref, w_ref, oa_ref, ob_ref):
        y = _dot_t(x_ref[...], w_ref[...])
        oa_ref[...] = y[:, :DH]
        ob_ref[...] = y[:, DH:]

    return pl.pallas_call(
        body,
        grid=_GRID,
        in_specs=[pl.BlockSpec((_BLK, D), lambda i: (i, 0)),
                  pl.BlockSpec((D, D), lambda i: (0, 0))],
        out_specs=[pl.BlockSpec((_BLK, DH), lambda i: (i, 0)),
                   pl.BlockSpec((_BLK, DH), lambda i: (i, 0))],
        out_shape=[jax.ShapeDtypeStruct((N_NODES, DH), jnp.float32)] * 2,
    )(X, W)


def _epilogue(ya, yb, b, g, beta):
    h = jnp.concatenate([ya, yb], axis=1) + b
    h = jnp.where(h >= 0.0, h, 0.01 * h)
    mu = jnp.mean(h, axis=1, keepdims=True)
    d = h - mu
    var = jnp.mean(d * d, axis=1, keepdims=True)
    return d * lax.rsqrt(var + 1e-5) * g + beta


def _tc_mid(ya, yb, b, g, beta, W):
    def body(ya_ref, yb_ref, b_ref, g_ref, be_ref, w_ref,
             h_ref, oa_ref, ob_ref):
        hn = _epilogue(ya_ref[...], yb_ref[...],
                       b_ref[...], g_ref[...], be_ref[...])
        h_ref[...] = hn
        y = _dot_t(hn, w_ref[...])
        oa_ref[...] = y[:, :DH]
        ob_ref[...] = y[:, DH:]

    vec = pl.BlockSpec((1, D), lambda i: (0, 0))
    return pl.pallas_call(
        body,
        grid=_GRID,
        in_specs=[pl.BlockSpec((_BLK, DH), lambda i: (i, 0)),
                  pl.BlockSpec((_BLK, DH), lambda i: (i, 0)),
                  vec, vec, vec,
                  pl.BlockSpec((D, D), lambda i: (0, 0))],
        out_specs=[pl.BlockSpec((_BLK, D), lambda i: (i, 0)),
                   pl.BlockSpec((_BLK, DH), lambda i: (i, 0)),
                   pl.BlockSpec((_BLK, DH), lambda i: (i, 0))],
        out_shape=[jax.ShapeDtypeStruct((N_NODES, D), jnp.float32),
                   jax.ShapeDtypeStruct((N_NODES, DH), jnp.float32),
                   jax.ShapeDtypeStruct((N_NODES, DH), jnp.float32)],
    )(ya, yb, b, g, beta, W)


def _tc_final(ya, yb, b, g, beta, X, h1):
    def body(ya_ref, yb_ref, b_ref, g_ref, be_ref, x_ref, h1_ref, o_ref):
        h2 = _epilogue(ya_ref[...], yb_ref[...],
                       b_ref[...], g_ref[...], be_ref[...])
        o_ref[...] = (x_ref[...] + h1_ref[...] + h2) * (1.0 / 3.0)

    vec = pl.BlockSpec((1, D), lambda i: (0, 0))
    return pl.pallas_call(
        body,
        grid=_GRID,
        in_specs=[pl.BlockSpec((_BLK, DH), lambda i: (i, 0)),
                  pl.BlockSpec((_BLK, DH), lambda i: (i, 0)),
                  vec, vec, vec,
                  pl.BlockSpec((_BLK, D), lambda i: (i, 0)),
                  pl.BlockSpec((_BLK, D), lambda i: (i, 0))],
        out_specs=pl.BlockSpec((_BLK, D), lambda i: (i, 0)),
        out_shape=jax.ShapeDtypeStruct((N_NODES, D), jnp.float32),
    )(ya, yb, b, g, beta, X, h1)


def kernel(X, A, W0, b0, g0, beta0, W1, b1, g1, beta1):
    nidx = A[0]
    eidx = A[1]
    b0r, g0r, be0r = (v.reshape(1, D) for v in (b0, g0, beta0))
    b1r, g1r, be1r = (v.reshape(1, D) for v in (b1, g1, beta1))

    xa0, xb0 = _tc_lin(X, W0)
    oa0, ob0, ndeg, edeg = _sc_layer0(xa0, xb0, nidx, eidx)
    h1, xa1, xb1 = _tc_mid(oa0, ob0, b0r, g0r, be0r, W1)
    oa1, ob1 = _sc_layer1(xa1, xb1, nidx, eidx, ndeg, edeg)
    return _tc_final(oa1, ob1, b1r, g1r, be1r, X, h1)


# speedup vs baseline: 5.2495x; 1.2054x over previous
import functools

import jax
import jax.numpy as jnp
from jax import lax
from jax.experimental import pallas as pl
from jax.experimental.pallas import tpu as pltpu
from jax.experimental.pallas import tpu_sc as plsc

N_NODES = 10000
N_EDGES = 10000
NNZ = 160000
D = 256
DH = 128
NC = 2
NS = 16
L = 16

NPAD = 10240
ROWS_PER_TILE = NPAD // NS
NNZ_PER_TILE = NNZ // NS
CH = 128
NNZ_PT_PAD = 10240
NCHUNK = NNZ_PT_PAD // CH
RCH = 128
NRCH = ROWS_PER_TILE // RCH
JUNK = N_NODES

_MESH = plsc.VectorSubcoreMesh(
    core_axis_name="c", subcore_axis_name="s", num_cores=NC, num_subcores=NS)


def _make_sc_layer(compute_degs):
    out_type = [jax.ShapeDtypeStruct((NPAD, DH), jnp.float32),
                jax.ShapeDtypeStruct((NPAD, DH), jnp.float32)]
    if compute_degs:
        out_type += [jax.ShapeDtypeStruct((NC, NPAD, DH), jnp.float32),
                     jax.ShapeDtypeStruct((NC, NPAD, DH), jnp.float32)]
    scratch = [
        pltpu.VMEM_SHARED((NPAD, DH), jnp.float32),
        pltpu.VMEM((2, CH), jnp.int32),
        pltpu.VMEM((2, CH), jnp.int32),
        pltpu.VMEM((CH, DH), jnp.float32),
        pltpu.VMEM((CH, DH), jnp.float32),
        pltpu.SemaphoreType.DMA,
        pltpu.SemaphoreType.DMA,
        pltpu.SemaphoreType.DMA,
        pltpu.SemaphoreType.DMA,
    ]
    NH = NCHUNK // 2

    def body(*refs):
        if compute_degs:
            (xl_a, xl_b, pairs1, pairs2,
             out_a, out_b, ndeg_out, edeg_out,
             acc, pairA, pairB, bufA, bufB, gsA, gsB, isA, isB) = refs
            ndeg, edeg = ndeg_out, edeg_out
        else:
            (xl_a, xl_b, pairs1, pairs2, ndeg_in, edeg_in,
             out_a, out_b,
             acc, pairA, pairB, bufA, bufB, gsA, gsB, isA, isB) = refs
            ndeg, edeg = ndeg_in, edeg_in
        c = lax.axis_index("c")
        s = lax.axis_index("s")
        p1 = pairs1.at[s]
        p2 = pairs2.at[s]

        def fill_const(ref, val):
            v = jnp.full((L,), val, jnp.float32)

            def fb(r, carry):
                for j in range(DH // L):
                    ref[r, pl.ds(j * L, L)] = v
                return carry
            lax.fori_loop(0, CH, fb, 0)

        def zero_acc():
            fill_const(bufB, 0.0)
            for k in range(NRCH):
                pltpu.sync_copy(
                    bufB, acc.at[pl.ds(s * ROWS_PER_TILE + k * RCH, RCH)])

        def hist_pass(ps):
            pltpu.sync_copy(ps.at[0], pairA)
            pltpu.async_copy(ps.at[1], pairB, isB)

            def hb(j, carry):
                @pl.when(j > 0)
                def _():
                    pltpu.make_async_copy(ps.at[0], pairA, isA).wait()
                pltpu.sync_copy(bufA, acc.at[pairA.at[1]], add=True)

                @pl.when(j < NH - 1)
                def _():
                    pltpu.async_copy(ps.at[2 * j + 2], pairA, isA)
                pltpu.make_async_copy(ps.at[0], pairB, isB).wait()
                pltpu.sync_copy(bufA, acc.at[pairB.at[1]], add=True)

                @pl.when(j < NH - 1)
                def _():
                    pltpu.async_copy(ps.at[2 * j + 3], pairB, isB)
                return carry
            lax.fori_loop(0, NH, hb, 0)

        def dump_deg(table):
            for k in range(NRCH):
                sl = pl.ds(s * ROWS_PER_TILE + k * RCH, RCH)
                pltpu.sync_copy(acc.at[sl], bufB)
                pltpu.sync_copy(bufB, table.at[c].at[sl])

        def seg_pass(src_hbm, ps):
            pltpu.sync_copy(ps.at[0], pairA)
            pltpu.async_copy(src_hbm.at[pairA.at[0]], bufA, gsA)
            pltpu.async_copy(ps.at[1], pairB, isB)

            def sb(j, carry):
                pltpu.make_async_copy(ps.at[0], pairB, isB).wait()
                pltpu.async_copy(src_hbm.at[pairB.at[0]], bufB, gsB)
                pltpu.make_async_copy(src_hbm.at[pairA.at[0]], bufA, gsA).wait()
                pltpu.sync_copy(bufA, acc.at[pairA.at[1]], add=True)

                @pl.when(j < NH - 1)
                def _():
                    pltpu.async_copy(ps.at[2 * j + 2], pairA, isA)
                    pltpu.make_async_copy(ps.at[0], pairA, isA).wait()
                    pltpu.async_copy(src_hbm.at[pairA.at[0]], bufA, gsA)
                pltpu.make_async_copy(src_hbm.at[pairB.at[0]], bufB, gsB).wait()
                pltpu.sync_copy(bufB, acc.at[pairB.at[1]], add=True)

                @pl.when(j < NH - 1)
                def _():
                    pltpu.async_copy(ps.at[2 * j + 3], pairB, isB)
                return carry
            lax.fori_loop(0, NH, sb, 0)

        def scale_write(table, out_hbm):
            for k in range(NRCH):
                sl = pl.ds(s * ROWS_PER_TILE + k * RCH, RCH)
                pltpu.sync_copy(acc.at[sl], bufA)
                pltpu.sync_copy(table.at[c].at[sl], bufB)

                def rb(r, carry):
                    dv = bufB[r, pl.ds(0, L)]
                    inv = jnp.where(dv > 0.0, 1.0 / dv, 0.0)
                    for j in range(DH // L):
                        cs = pl.ds(j * L, L)
                        bufA[r, cs] = bufA[r, cs] * inv
                    return carry
                lax.fori_loop(0, RCH, rb, 0)
                pltpu.sync_copy(bufA, out_hbm.at[sl])

        if compute_degs:
            fill_const(bufA, 1.0)
            zero_acc()
            plsc.subcore_barrier()
            hist_pass(p1)
            plsc.subcore_barrier()
            dump_deg(edeg)
            zero_acc()
            plsc.subcore_barrier()
            hist_pass(p2)
            plsc.subcore_barrier()
            dump_deg(ndeg)

        zero_acc()
        plsc.subcore_barrier()

        def run_half(xl, out_h):
            seg_pass(xl, p1)
            plsc.subcore_barrier()
            scale_write(edeg, out_h)
            zero_acc()
            plsc.subcore_barrier()
            seg_pass(out_h, p2)
            plsc.subcore_barrier()
            scale_write(ndeg, out_h)

        @pl.when(c == 0)
        def _():
            run_half(xl_a, out_a)

        @pl.when(c == 1)
        def _():
            run_half(xl_b, out_b)

    return pl.kernel(body, out_type=out_type, mesh=_MESH,
                     scratch_types=scratch)


_sc_layer0 = _make_sc_layer(True)
_sc_layer1 = _make_sc_layer(False)


_BLK = 1000
_GRID = (N_NODES // _BLK,)


def _dot_t(x, w):
    return lax.dot_general(x, w, (((1,), (1,)), ((), ())),
                           preferred_element_type=jnp.float32)


def _tc_lin(X, W):
    def body(x_ref, w_ref, oa_ref, ob_ref):
        y = _dot_t(x_ref[...], w_ref[...])
        oa_ref[...] = y[:, :DH]
        ob_ref[...] = y[:, DH:]

    return pl.pallas_call(
        body,
        grid=_GRID,
        in_specs=[pl.BlockSpec((_BLK, D), lambda i: (i, 0)),
                  pl.BlockSpec((D, D), lambda i: (0, 0))],
        out_specs=[pl.BlockSpec((_BLK, DH), lambda i: (i, 0)),
                   pl.BlockSpec((_BLK, DH), lambda i: (i, 0))],
        out_shape=[jax.ShapeDtypeStruct((N_NODES, DH), jnp.float32)] * 2,
    )(X, W)


def _epilogue(ya, yb, b, g, beta):
    h = jnp.concatenate([ya, yb], axis=1) + b
    h = jnp.where(h >= 0.0, h, 0.01 * h)
    mu = jnp.mean(h, axis=1, keepdims=True)
    d = h - mu
    var = jnp.mean(d * d, axis=1, keepdims=True)
    return d * lax.rsqrt(var + 1e-5) * g + beta


def _tc_mid(ya, yb, b, g, beta, W):
    def body(ya_ref, yb_ref, b_ref, g_ref, be_ref, w_ref,
             h_ref, oa_ref, ob_ref):
        hn = _epilogue(ya_ref[...], yb_ref[...],
                       b_ref[...], g_ref[...], be_ref[...])
        h_ref[...] = hn
        y = _dot_t(hn, w_ref[...])
        oa_ref[...] = y[:, :DH]
        ob_ref[...] = y[:, DH:]

    vec = pl.BlockSpec((1, D), lambda i: (0, 0))
    return pl.pallas_call(
        body,
        grid=_GRID,
        in_specs=[pl.BlockSpec((_BLK, DH), lambda i: (i, 0)),
                  pl.BlockSpec((_BLK, DH), lambda i: (i, 0)),
                  vec, vec, vec,
                  pl.BlockSpec((D, D), lambda i: (0, 0))],
        out_specs=[pl.BlockSpec((_BLK, D), lambda i: (i, 0)),
                   pl.BlockSpec((_BLK, DH), lambda i: (i, 0)),
                   pl.BlockSpec((_BLK, DH), lambda i: (i, 0))],
        out_shape=[jax.ShapeDtypeStruct((N_NODES, D), jnp.float32),
                   jax.ShapeDtypeStruct((N_NODES, DH), jnp.float32),
                   jax.ShapeDtypeStruct((N_NODES, DH), jnp.float32)],
    )(ya, yb, b, g, beta, W)


def _tc_final(ya, yb, b, g, beta, X, h1):
    def body(ya_ref, yb_ref, b_ref, g_ref, be_ref, x_ref, h1_ref, o_ref):
        h2 = _epilogue(ya_ref[...], yb_ref[...],
                       b_ref[...], g_ref[...], be_ref[...])
        o_ref[...] = (x_ref[...] + h1_ref[...] + h2) * (1.0 / 3.0)

    vec = pl.BlockSpec((1, D), lambda i: (0, 0))
    return pl.pallas_call(
        body,
        grid=_GRID,
        in_specs=[pl.BlockSpec((_BLK, DH), lambda i: (i, 0)),
                  pl.BlockSpec((_BLK, DH), lambda i: (i, 0)),
                  vec, vec, vec,
                  pl.BlockSpec((_BLK, D), lambda i: (i, 0)),
                  pl.BlockSpec((_BLK, D), lambda i: (i, 0))],
        out_specs=pl.BlockSpec((_BLK, D), lambda i: (i, 0)),
        out_shape=jax.ShapeDtypeStruct((N_NODES, D), jnp.float32),
    )(ya, yb, b, g, beta, X, h1)


def kernel(X, A, W0, b0, g0, beta0, W1, b1, g1, beta1):
    pad = ((0, 0), (0, NNZ_PT_PAD - NNZ_PER_TILE))
    n2 = A[0].reshape(NS, NNZ_PER_TILE)
    e2 = A[1].reshape(NS, NNZ_PER_TILE)
    n0 = jnp.pad(n2, pad).reshape(NS, NCHUNK, CH)
    nj = jnp.pad(n2, pad, constant_values=JUNK).reshape(NS, NCHUNK, CH)
    ej = jnp.pad(e2, pad, constant_values=JUNK).reshape(NS, NCHUNK, CH)
    pairs1 = jnp.stack([n0, ej], axis=2)
    pairs2 = jnp.stack([ej, nj], axis=2)
    b0r, g0r, be0r = (v.reshape(1, D) for v in (b0, g0, beta0))
    b1r, g1r, be1r = (v.reshape(1, D) for v in (b1, g1, beta1))

    xa0, xb0 = _tc_lin(X, W0)
    oa0, ob0, ndeg, edeg = _sc_layer0(xa0, xb0, pairs1, pairs2)
    h1, xa1, xb1 = _tc_mid(oa0, ob0, b0r, g0r, be0r, W1)
    oa1, ob1 = _sc_layer1(xa1, xb1, pairs1, pairs2, ndeg, edeg)
    return _tc_final(oa1, ob1, b1r, g1r, be1r, X, h1)


# speedup vs baseline: 5.5856x; 1.0640x over previous
import functools

import jax
import jax.numpy as jnp
from jax import lax
from jax.experimental import pallas as pl
from jax.experimental.pallas import tpu as pltpu
from jax.experimental.pallas import tpu_sc as plsc

N_NODES = 10000
N_EDGES = 10000
NNZ = 160000
D = 256
DH = 128
NC = 2
NS = 16
L = 16

NPAD = 10240
ROWS_PER_TILE = NPAD // NS
NNZ_PER_TILE = NNZ // NS
CH = 128
NNZ_PT_PAD = 10240
NCHUNK = NNZ_PT_PAD // CH
RCH = 128
NRCH = ROWS_PER_TILE // RCH
JUNK = N_NODES

_MESH = plsc.VectorSubcoreMesh(
    core_axis_name="c", subcore_axis_name="s", num_cores=NC, num_subcores=NS)


def _make_sc_kernels():
    out_type = [jax.ShapeDtypeStruct((NPAD, DH), jnp.float32),
                jax.ShapeDtypeStruct((NPAD, DH), jnp.float32)]
    deg_out_type = [jax.ShapeDtypeStruct((NPAD, DH), jnp.float32),
                    jax.ShapeDtypeStruct((NPAD, DH), jnp.float32)]
    scratch = [
        pltpu.VMEM_SHARED((NPAD, DH), jnp.float32),
        pltpu.VMEM((2, CH), jnp.int32),
        pltpu.VMEM((2, CH), jnp.int32),
        pltpu.VMEM((CH, DH), jnp.float32),
        pltpu.VMEM((CH, DH), jnp.float32),
        pltpu.SemaphoreType.DMA,
        pltpu.SemaphoreType.DMA,
        pltpu.SemaphoreType.DMA,
        pltpu.SemaphoreType.DMA,
    ]
    NH = NCHUNK // 2

    def _common(refs):
        (acc, pairA, pairB, bufA, bufB, gsA, gsB, isA, isB) = refs
        c = lax.axis_index("c")
        s = lax.axis_index("s")

        def fill_const(ref, val):
            v = jnp.full((L,), val, jnp.float32)

            def fb(r, carry):
                for j in range(DH // L):
                    ref[r, pl.ds(j * L, L)] = v
                return carry
            lax.fori_loop(0, CH, fb, 0)

        def zero_acc():
            fill_const(bufB, 0.0)
            for k in range(NRCH):
                pltpu.sync_copy(
                    bufB, acc.at[pl.ds(s * ROWS_PER_TILE + k * RCH, RCH)])

        def hist_pass(ps):
            pltpu.sync_copy(ps.at[0], pairA)
            pltpu.async_copy(ps.at[1], pairB, isB)

            def hb(j, carry):
                @pl.when(j > 0)
                def _():
                    pltpu.make_async_copy(ps.at[0], pairA, isA).wait()
                pltpu.sync_copy(bufA, acc.at[pairA.at[1]], add=True)

                @pl.when(j < NH - 1)
                def _():
                    pltpu.async_copy(ps.at[2 * j + 2], pairA, isA)
                pltpu.make_async_copy(ps.at[0], pairB, isB).wait()
                pltpu.sync_copy(bufA, acc.at[pairB.at[1]], add=True)

                @pl.when(j < NH - 1)
                def _():
                    pltpu.async_copy(ps.at[2 * j + 3], pairB, isB)
                return carry
            lax.fori_loop(0, NH, hb, 0)

        def dump_deg(table):
            for k in range(NRCH):
                sl = pl.ds(s * ROWS_PER_TILE + k * RCH, RCH)
                pltpu.sync_copy(acc.at[sl], bufB)
                pltpu.sync_copy(bufB, table.at[sl])

        def seg_pass(src_hbm, ps):
            pltpu.sync_copy(ps.at[0], pairA)
            pltpu.async_copy(src_hbm.at[pairA.at[0]], bufA, gsA)
            pltpu.async_copy(ps.at[1], pairB, isB)

            def sb(j, carry):
                pltpu.make_async_copy(ps.at[0], pairB, isB).wait()
                pltpu.async_copy(src_hbm.at[pairB.at[0]], bufB, gsB)
                pltpu.make_async_copy(src_hbm.at[pairA.at[0]], bufA, gsA).wait()
                pltpu.sync_copy(bufA, acc.at[pairA.at[1]], add=True)

                @pl.when(j < NH - 1)
                def _():
                    pltpu.async_copy(ps.at[2 * j + 2], pairA, isA)
                    pltpu.make_async_copy(ps.at[0], pairA, isA).wait()
                    pltpu.async_copy(src_hbm.at[pairA.at[0]], bufA, gsA)
                pltpu.make_async_copy(src_hbm.at[pairB.at[0]], bufB, gsB).wait()
                pltpu.sync_copy(bufB, acc.at[pairB.at[1]], add=True)

                @pl.when(j < NH - 1)
                def _():
                    pltpu.async_copy(ps.at[2 * j + 3], pairB, isB)
                return carry
            lax.fori_loop(0, NH, sb, 0)

        def scale_write(table, out_hbm):
            for k in range(NRCH):
                sl = pl.ds(s * ROWS_PER_TILE + k * RCH, RCH)
                pltpu.sync_copy(acc.at[sl], bufA)
                pltpu.sync_copy(table.at[sl], bufB)

                def rb(r, carry):
                    dv = bufB[r, pl.ds(0, L)]
                    inv = jnp.where(dv > 0.0, 1.0 / dv, 0.0)
                    for j in range(DH // L):
                        cs = pl.ds(j * L, L)
                        bufA[r, cs] = bufA[r, cs] * inv
                    return carry
                lax.fori_loop(0, RCH, rb, 0)
                pltpu.sync_copy(bufA, out_hbm.at[sl])

        return (c, s, fill_const, zero_acc, hist_pass, dump_deg,
                seg_pass, scale_write)

    def deg_body(pairs1, pairs2, ndeg_out, edeg_out, *refs):
        (c, s, fill_const, zero_acc, hist_pass, dump_deg,
         seg_pass, scale_write) = _common(refs)
        p1 = pairs1.at[s]
        p2 = pairs2.at[s]
        fill_const(refs[3], 1.0)
        zero_acc()
        plsc.subcore_barrier()

        @pl.when(c == 0)
        def _():
            hist_pass(p1)
        @pl.when(c == 1)
        def _():
            hist_pass(p2)
        plsc.subcore_barrier()

        @pl.when(c == 0)
        def _():
            dump_deg(edeg_out)
        @pl.when(c == 1)
        def _():
            dump_deg(ndeg_out)

    def layer_body(xl_a, xl_b, pairs1, pairs2, ndeg, edeg,
                   out_a, out_b, *refs):
        (c, s, fill_const, zero_acc, hist_pass, dump_deg,
         seg_pass, scale_write) = _common(refs)
        p1 = pairs1.at[s]
        p2 = pairs2.at[s]
        zero_acc()
        plsc.subcore_barrier()

        def run_half(xl, out_h):
            seg_pass(xl, p1)
            plsc.subcore_barrier()
            scale_write(edeg, out_h)
            zero_acc()
            plsc.subcore_barrier()
            seg_pass(out_h, p2)
            plsc.subcore_barrier()
            scale_write(ndeg, out_h)

        @pl.when(c == 0)
        def _():
            run_half(xl_a, out_a)

        @pl.when(c == 1)
        def _():
            run_half(xl_b, out_b)

    deg_k = pl.kernel(deg_body, out_type=deg_out_type, mesh=_MESH,
                      scratch_types=scratch)
    layer_k = pl.kernel(layer_body, out_type=out_type, mesh=_MESH,
                        scratch_types=scratch)
    return deg_k, layer_k


_sc_degrees, _sc_layer = _make_sc_kernels()


_BLK = 1000
_GRID = (N_NODES // _BLK,)


def _dot_t(x, w):
    return lax.dot_general(x, w, (((1,), (1,)), ((), ())),
                           preferred_element_type=jnp.float32)


def _tc_lin(X, W):
    def body(x_ref, w_ref, oa_ref, ob_ref):
        y = _dot_t(x_ref[...], w_ref[...])
        oa_ref[...] = y[:, :DH]
        ob_ref[...] = y[:, DH:]

    return pl.pallas_call(
        body,
        grid=_GRID,
        in_specs=[pl.BlockSpec((_BLK, D), lambda i: (i, 0)),
                  pl.BlockSpec((D, D), lambda i: (0, 0))],
        out_specs=[pl.BlockSpec((_BLK, DH), lambda i: (i, 0)),
                   pl.BlockSpec((_BLK, DH), lambda i: (i, 0))],
        out_shape=[jax.ShapeDtypeStruct((N_NODES, DH), jnp.float32)] * 2,
    )(X, W)


def _epilogue(ya, yb, b, g, beta):
    h = jnp.concatenate([ya, yb], axis=1) + b
    h = jnp.where(h >= 0.0, h, 0.01 * h)
    mu = jnp.mean(h, axis=1, keepdims=True)
    d = h - mu
    var = jnp.mean(d * d, axis=1, keepdims=True)
    return d * lax.rsqrt(var + 1e-5) * g + beta


def _tc_mid(ya, yb, b, g, beta, W):
    def body(ya_ref, yb_ref, b_ref, g_ref, be_ref, w_ref,
             h_ref, oa_ref, ob_ref):
        hn = _epilogue(ya_ref[...], yb_ref[...],
                       b_ref[...], g_ref[...], be_ref[...])
        h_ref[...] = hn
        y = _dot_t(hn, w_ref[...])
        oa_ref[...] = y[:, :DH]
        ob_ref[...] = y[:, DH:]

    vec = pl.BlockSpec((1, D), lambda i: (0, 0))
    return pl.pallas_call(
        body,
        grid=_GRID,
        in_specs=[pl.BlockSpec((_BLK, DH), lambda i: (i, 0)),
                  pl.BlockSpec((_BLK, DH), lambda i: (i, 0)),
                  vec, vec, vec,
                  pl.BlockSpec((D, D), lambda i: (0, 0))],
        out_specs=[pl.BlockSpec((_BLK, D), lambda i: (i, 0)),
                   pl.BlockSpec((_BLK, DH), lambda i: (i, 0)),
                   pl.BlockSpec((_BLK, DH), lambda i: (i, 0))],
        out_shape=[jax.ShapeDtypeStruct((N_NODES, D), jnp.float32),
                   jax.ShapeDtypeStruct((N_NODES, DH), jnp.float32),
                   jax.ShapeDtypeStruct((N_NODES, DH), jnp.float32)],
    )(ya, yb, b, g, beta, W)


def _tc_final(ya, yb, b, g, beta, X, h1):
    def body(ya_ref, yb_ref, b_ref, g_ref, be_ref, x_ref, h1_ref, o_ref):
        h2 = _epilogue(ya_ref[...], yb_ref[...],
                       b_ref[...], g_ref[...], be_ref[...])
        o_ref[...] = (x_ref[...] + h1_ref[...] + h2) * (1.0 / 3.0)

    vec = pl.BlockSpec((1, D), lambda i: (0, 0))
    return pl.pallas_call(
        body,
        grid=_GRID,
        in_specs=[pl.BlockSpec((_BLK, DH), lambda i: (i, 0)),
                  pl.BlockSpec((_BLK, DH), lambda i: (i, 0)),
                  vec, vec, vec,
                  pl.BlockSpec((_BLK, D), lambda i: (i, 0)),
                  pl.BlockSpec((_BLK, D), lambda i: (i, 0))],
        out_specs=pl.BlockSpec((_BLK, D), lambda i: (i, 0)),
        out_shape=jax.ShapeDtypeStruct((N_NODES, D), jnp.float32),
    )(ya, yb, b, g, beta, X, h1)


def kernel(X, A, W0, b0, g0, beta0, W1, b1, g1, beta1):
    pad = ((0, 0), (0, NNZ_PT_PAD - NNZ_PER_TILE))
    n2 = A[0].reshape(NS, NNZ_PER_TILE)
    e2 = A[1].reshape(NS, NNZ_PER_TILE)
    n0 = jnp.pad(n2, pad).reshape(NS, NCHUNK, CH)
    nj = jnp.pad(n2, pad, constant_values=JUNK).reshape(NS, NCHUNK, CH)
    ej = jnp.pad(e2, pad, constant_values=JUNK).reshape(NS, NCHUNK, CH)
    pairs1 = jnp.stack([n0, ej], axis=2)
    pairs2 = jnp.stack([ej, nj], axis=2)
    b0r, g0r, be0r = (v.reshape(1, D) for v in (b0, g0, beta0))
    b1r, g1r, be1r = (v.reshape(1, D) for v in (b1, g1, beta1))

    ndeg, edeg = _sc_degrees(pairs1, pairs2)
    xa0, xb0 = _tc_lin(X, W0)
    oa0, ob0 = _sc_layer(xa0, xb0, pairs1, pairs2, ndeg, edeg)
    h1, xa1, xb1 = _tc_mid(oa0, ob0, b0r, g0r, be0r, W1)
    oa1, ob1 = _sc_layer(xa1, xb1, pairs1, pairs2, ndeg, edeg)
    return _tc_final(oa1, ob1, b1r, g1r, be1r, X, h1)
